# baseline (speedup 1.0000x reference)
import functools

import jax
import jax.numpy as jnp
from jax import lax
from jax.experimental import pallas as pl
from jax.experimental.pallas import tpu as pltpu
from jax.experimental.pallas import tpu_sc as plsc

B = 1024
T = 50
N = B * T
V = 100000
D0 = 300
D1 = 100
OUT = 300
OUTP = 384

NC, NS = 2, 16
NW = NC * NS
TP = 56
CHUNK = TP
NCHUNK = B // NW

VB = 2048


def _table_body(e0t_ref, e1t_ref, W0p_ref, W1p_ref, b0p_ref, b1p_ref,
                wa_ref, out_ref):
    cdims = (((0,), (0,)), ((), ()))
    lastdims = (((1,), (1,)), ((), ()))
    p0 = lax.dot_general(e0t_ref[...], W0p_ref[...], cdims,
                         preferred_element_type=jnp.float32) + b0p_ref[...]
    p1 = lax.dot_general(e1t_ref[...], W1p_ref[...], cdims,
                         preferred_element_type=jnp.float32) + b1p_ref[...]
    wa = wa_ref[...]
    s0 = jnp.sum(p0 * wa, axis=1, keepdims=True)
    s1 = jnp.sum(p1 * wa, axis=1, keepdims=True)
    m = jnp.maximum(s0, s1)
    w0 = jnp.exp(s0 - m)
    w1 = jnp.exp(s1 - m)
    inv = 1.0 / (w0 + w1)
    out_ref[...] = (w0 * p0 + w1 * p1) * inv


def _tc_combined_table(emb0t, emb1t, W0p, W1p, b0p, b1p, wap):
    grid = (pl.cdiv(V, VB),)
    return pl.pallas_call(
        _table_body,
        grid=grid,
        in_specs=[
            pl.BlockSpec((D0, VB), lambda i: (0, i)),
            pl.BlockSpec((D1, VB), lambda i: (0, i)),
            pl.BlockSpec((D0, OUTP), lambda i: (0, 0)),
            pl.BlockSpec((D1, OUTP), lambda i: (0, 0)),
            pl.BlockSpec((1, OUTP), lambda i: (0, 0)),
            pl.BlockSpec((1, OUTP), lambda i: (0, 0)),
            pl.BlockSpec((1, OUTP), lambda i: (0, 0)),
        ],
        out_specs=pl.BlockSpec((VB, OUTP), lambda i: (i, 0)),
        out_shape=jax.ShapeDtypeStruct((V, OUTP), jnp.float32),
    )(emb0t, emb1t, W0p, W1p, b0p, b1p, wap)


def _sc_gather_body(idx_hbm, c_hbm, g_out, idx_v, buf, s0, s1):
    wid = lax.axis_index("s") * NC + lax.axis_index("c")
    pltpu.sync_copy(idx_hbm.at[wid], idx_v)
    base = wid * NCHUNK
    sems = (s0, s1)
    cps = [pltpu.async_copy(c_hbm.at[idx_v.at[0]], buf.at[0], s0), None]
    for j in range(NCHUNK):
        s = j & 1
        if j + 1 < NCHUNK:
            cps[1 - s] = pltpu.async_copy(
                c_hbm.at[idx_v.at[j + 1]], buf.at[1 - s], sems[1 - s])
        cps[s].wait()
        pltpu.sync_copy(buf.at[s], g_out.at[pl.ds((base + j) * TP, TP)])


def _sc_gather(idx, c):
    mesh = plsc.VectorSubcoreMesh(core_axis_name="c", subcore_axis_name="s")
    fn = pl.kernel(
        _sc_gather_body,
        mesh=mesh,
        out_type=jax.ShapeDtypeStruct((B * TP, OUTP), jnp.float32),
        scratch_types=[
            pltpu.VMEM((NCHUNK, CHUNK), jnp.int32),
            pltpu.VMEM((2, CHUNK, OUTP), jnp.float32),
            pltpu.SemaphoreType.DMA,
            pltpu.SemaphoreType.DMA,
        ],
    )
    return fn(idx, c)


def kernel(inputs, emb0, emb1, W0, b0, W1, b1, Wa, ba):
    idxp = jnp.concatenate([inputs, inputs[:, : TP - T]], axis=1)
    idx = idxp.reshape(NW, NCHUNK, CHUNK).astype(jnp.int32)
    emb0t = emb0.T
    emb1t = emb1.T
    W0p = jnp.pad(W0, ((0, 0), (0, OUTP - OUT)))
    W1p = jnp.pad(W1, ((0, 0), (0, OUTP - OUT)))
    b0p = jnp.pad(b0, (0, OUTP - OUT)).reshape(1, OUTP)
    b1p = jnp.pad(b1, (0, OUTP - OUT)).reshape(1, OUTP)
    wap = jnp.pad(Wa.reshape(1, OUT), ((0, 0), (0, OUTP - OUT)))
    c = _tc_combined_table(emb0t, emb1t, W0p, W1p, b0p, b1p, wap)
    g = _sc_gather(idx, c)
    return g.reshape(B, TP, OUTP)[:, :T, :OUT]


# speedup vs baseline: 1.0450x; 1.0450x over previous
import functools

import jax
import jax.numpy as jnp
from jax import lax
from jax.experimental import pallas as pl
from jax.experimental.pallas import tpu as pltpu
from jax.experimental.pallas import tpu_sc as plsc

B = 1024
T = 50
N = B * T
V = 100000
D0 = 300
D1 = 100
OUT = 300
OUTP = 384

NC, NS = 2, 16
NW = NC * NS
TP = 56
CHUNK = 2 * TP
NCHUNK = B // NW // 2

VB = 4096


def _table_body(e0t_ref, e1t_ref, W0p_ref, W1p_ref, b0p_ref, b1p_ref,
                wa_ref, out_ref):
    cdims = (((0,), (0,)), ((), ()))
    lastdims = (((1,), (1,)), ((), ()))
    p0 = lax.dot_general(e0t_ref[...], W0p_ref[...], cdims,
                         preferred_element_type=jnp.float32) + b0p_ref[...]
    p1 = lax.dot_general(e1t_ref[...], W1p_ref[...], cdims,
                         preferred_element_type=jnp.float32) + b1p_ref[...]
    wa = wa_ref[...]
    s0 = jnp.sum(p0 * wa, axis=1, keepdims=True)
    s1 = jnp.sum(p1 * wa, axis=1, keepdims=True)
    m = jnp.maximum(s0, s1)
    w0 = jnp.exp(s0 - m)
    w1 = jnp.exp(s1 - m)
    inv = 1.0 / (w0 + w1)
    out_ref[...] = (w0 * p0 + w1 * p1) * inv


def _tc_combined_table(emb0t, emb1t, W0p, W1p, b0p, b1p, wap):
    grid = (pl.cdiv(V, VB),)
    return pl.pallas_call(
        _table_body,
        grid=grid,
        in_specs=[
            pl.BlockSpec((D0, VB), lambda i: (0, i)),
            pl.BlockSpec((D1, VB), lambda i: (0, i)),
            pl.BlockSpec((D0, OUTP), lambda i: (0, 0)),
            pl.BlockSpec((D1, OUTP), lambda i: (0, 0)),
            pl.BlockSpec((1, OUTP), lambda i: (0, 0)),
            pl.BlockSpec((1, OUTP), lambda i: (0, 0)),
            pl.BlockSpec((1, OUTP), lambda i: (0, 0)),
        ],
        out_specs=pl.BlockSpec((VB, OUTP), lambda i: (i, 0)),
        out_shape=jax.ShapeDtypeStruct((V, OUTP), jnp.float32),
    )(emb0t, emb1t, W0p, W1p, b0p, b1p, wap)


def _sc_gather_body(idx_hbm, c_hbm, g_out, idx_v, buf, s0, s1):
    wid = lax.axis_index("s") * NC + lax.axis_index("c")
    pltpu.sync_copy(idx_hbm.at[wid], idx_v)
    base = wid * NCHUNK
    sems = (s0, s1)
    cps = [pltpu.async_copy(c_hbm.at[idx_v.at[0]], buf.at[0], s0), None]
    for j in range(NCHUNK):
        s = j & 1
        if j + 1 < NCHUNK:
            cps[1 - s] = pltpu.async_copy(
                c_hbm.at[idx_v.at[j + 1]], buf.at[1 - s], sems[1 - s])
        cps[s].wait()
        pltpu.sync_copy(buf.at[s], g_out.at[pl.ds((base + j) * CHUNK, CHUNK)])


def _sc_gather(idx, c):
    mesh = plsc.VectorSubcoreMesh(core_axis_name="c", subcore_axis_name="s")
    fn = pl.kernel(
        _sc_gather_body,
        mesh=mesh,
        out_type=jax.ShapeDtypeStruct((B * TP, OUTP), jnp.float32),
        scratch_types=[
            pltpu.VMEM((NCHUNK, CHUNK), jnp.int32),
            pltpu.VMEM((2, CHUNK, OUTP), jnp.float32),
            pltpu.SemaphoreType.DMA,
            pltpu.SemaphoreType.DMA,
        ],
    )
    return fn(idx, c)


def kernel(inputs, emb0, emb1, W0, b0, W1, b1, Wa, ba):
    idxp = jnp.concatenate([inputs, inputs[:, : TP - T]], axis=1)
    idx = idxp.reshape(NW, NCHUNK, CHUNK).astype(jnp.int32)
    emb0t = emb0.T
    emb1t = emb1.T
    W0p = jnp.pad(W0, ((0, 0), (0, OUTP - OUT)))
    W1p = jnp.pad(W1, ((0, 0), (0, OUTP - OUT)))
    b0p = jnp.pad(b0, (0, OUTP - OUT)).reshape(1, OUTP)
    b1p = jnp.pad(b1, (0, OUTP - OUT)).reshape(1, OUTP)
    wap = jnp.pad(Wa.reshape(1, OUT), ((0, 0), (0, OUTP - OUT)))
    c = _tc_combined_table(emb0t, emb1t, W0p, W1p, b0p, b1p, wap)
    g = _sc_gather(idx, c)
    return g.reshape(B, TP, OUTP)[:, :T, :OUT]


# speedup vs baseline: 1.0539x; 1.0086x over previous
import functools

import jax
import jax.numpy as jnp
from jax import lax
from jax.experimental import pallas as pl
from jax.experimental.pallas import tpu as pltpu
from jax.experimental.pallas import tpu_sc as plsc

B = 1024
T = 50
N = B * T
V = 100000
D0 = 300
D1 = 100
OUT = 300
OUTP = 384

NC, NS = 2, 16
NW = NC * NS
TP = 56
CHUNK = 2 * TP
NCHUNK = B // NW // 2

VB = 4096


def _table_body(e0t_ref, e1t_ref, W0_ref, W1_ref, b0_ref, b1_ref,
                wa_ref, out_ref):
    cdims = (((0,), (0,)), ((), ()))
    p0 = lax.dot_general(e0t_ref[...], W0_ref[...], cdims,
                         preferred_element_type=jnp.float32) + b0_ref[...]
    p1 = lax.dot_general(e1t_ref[...], W1_ref[...], cdims,
                         preferred_element_type=jnp.float32) + b1_ref[...]
    wa = wa_ref[...]
    s0 = jnp.sum(p0 * wa, axis=1, keepdims=True)
    s1 = jnp.sum(p1 * wa, axis=1, keepdims=True)
    m = jnp.maximum(s0, s1)
    w0 = jnp.exp(s0 - m)
    w1 = jnp.exp(s1 - m)
    inv = 1.0 / (w0 + w1)
    out_ref[:, :OUT] = (w0 * p0 + w1 * p1) * inv
    out_ref[:, OUT:] = jnp.zeros((VB, OUTP - OUT), jnp.float32)


def _tc_combined_table(emb0t, emb1t, W0, W1, b0r, b1r, war):
    grid = (pl.cdiv(V, VB),)
    return pl.pallas_call(
        _table_body,
        grid=grid,
        in_specs=[
            pl.BlockSpec((D0, VB), lambda i: (0, i)),
            pl.BlockSpec((D1, VB), lambda i: (0, i)),
            pl.BlockSpec((D0, OUT), lambda i: (0, 0)),
            pl.BlockSpec((D1, OUT), lambda i: (0, 0)),
            pl.BlockSpec((1, OUT), lambda i: (0, 0)),
            pl.BlockSpec((1, OUT), lambda i: (0, 0)),
            pl.BlockSpec((1, OUT), lambda i: (0, 0)),
        ],
        out_specs=pl.BlockSpec((VB, OUTP), lambda i: (i, 0)),
        out_shape=jax.ShapeDtypeStruct((V, OUTP), jnp.float32),
    )(emb0t, emb1t, W0, W1, b0r, b1r, war)


def _sc_gather_body(idx_hbm, c_hbm, g_out, idx_v, buf, s0, s1):
    wid = lax.axis_index("s") * NC + lax.axis_index("c")
    pltpu.sync_copy(idx_hbm.at[wid], idx_v)
    base = wid * NCHUNK
    sems = (s0, s1)
    cps = [pltpu.async_copy(c_hbm.at[idx_v.at[0]], buf.at[0], s0), None]
    for j in range(NCHUNK):
        s = j & 1
        if j + 1 < NCHUNK:
            cps[1 - s] = pltpu.async_copy(
                c_hbm.at[idx_v.at[j + 1]], buf.at[1 - s], sems[1 - s])
        cps[s].wait()
        pltpu.sync_copy(buf.at[s], g_out.at[pl.ds((base + j) * CHUNK, CHUNK)])


def _sc_gather(idx, c):
    mesh = plsc.VectorSubcoreMesh(core_axis_name="c", subcore_axis_name="s")
    fn = pl.kernel(
        _sc_gather_body,
        mesh=mesh,
        out_type=jax.ShapeDtypeStruct((B * TP, OUTP), jnp.float32),
        scratch_types=[
            pltpu.VMEM((NCHUNK, CHUNK), jnp.int32),
            pltpu.VMEM((2, CHUNK, OUTP), jnp.float32),
            pltpu.SemaphoreType.DMA,
            pltpu.SemaphoreType.DMA,
        ],
    )
    return fn(idx, c)


def kernel(inputs, emb0, emb1, W0, b0, W1, b1, Wa, ba):
    idxp = jnp.concatenate([inputs, inputs[:, : TP - T]], axis=1)
    idx = idxp.reshape(NW, NCHUNK, CHUNK).astype(jnp.int32)
    emb0t = emb0.T
    emb1t = emb1.T
    c = _tc_combined_table(emb0t, emb1t, W0, W1, b0.reshape(1, OUT),
                           b1.reshape(1, OUT), Wa.reshape(1, OUT))
    g = _sc_gather(idx, c)
    return g.reshape(B, TP, OUTP)[:, :T, :OUT]
